# baseline (speedup 1.0000x reference)
import functools

import jax
import jax.numpy as jnp
from jax import lax
from jax.experimental import pallas as pl
from jax.experimental.pallas import tpu as pltpu
from jax.experimental.pallas import tpu_sc as plsc

F = 64
CTX = 24
IN_DIM = 3 * F + CTX


@functools.lru_cache(maxsize=None)
def _build_rowdot(K, N, blk, w_lo):
    nblk = -(-N // blk)

    def body(w_ref, m_ref, o_ref):
        prod = jax.lax.dot_general(
            w_ref[:, w_lo:w_lo + K], m_ref[...], (((1,), (0,)), ((), ())),
            preferred_element_type=jnp.float32,
            precision=jax.lax.Precision.DEFAULT)
        o_ref[...] = prod.reshape(o_ref.shape)

    return pl.pallas_call(
        body,
        grid=(nblk,),
        in_specs=[
            pl.BlockSpec((1, IN_DIM), lambda j: (0, 0)),
            pl.BlockSpec((K, blk), lambda j: (0, j)),
        ],
        out_specs=pl.BlockSpec((blk,), lambda j: (j,)),
        out_shape=jax.ShapeDtypeStruct((nblk * blk,), jnp.float32),
    )


@functools.lru_cache(maxsize=None)
def _build_ctx_shift(n_info, B, n_shift):
    def body(w_ref, m_ref, st_ref, o_ref, sd_ref):
        prod = jax.lax.dot_general(
            w_ref[:, :n_info], m_ref[...], (((1,), (0,)), ((), ())),
            preferred_element_type=jnp.float32,
            precision=jax.lax.Precision.DEFAULT)
        o_ref[...] = prod.reshape(o_ref.shape)
        sdot = jax.lax.dot_general(
            w_ref[:, CTX:CTX + F], st_ref[...], (((1,), (0,)), ((), ())),
            preferred_element_type=jnp.float32,
            precision=jax.lax.Precision.DEFAULT)
        sd_ref[...] = jnp.pad(sdot.reshape(n_shift), (0, 128 - n_shift))

    return pl.pallas_call(
        body,
        in_specs=[
            pl.BlockSpec((1, IN_DIM), lambda: (0, 0)),
            pl.BlockSpec((n_info, B), lambda: (0, 0)),
            pl.BlockSpec((F, n_shift), lambda: (0, 0)),
        ],
        out_specs=[
            pl.BlockSpec((B,), lambda: (0,)),
            pl.BlockSpec((128,), lambda: (0,)),
        ],
        out_shape=[
            jax.ShapeDtypeStruct((B,), jnp.float32),
            jax.ShapeDtypeStruct((128,), jnp.float32),
        ],
    )


@functools.lru_cache(maxsize=None)
def _build_sc_combine(B):
    NC, NS = 2, 16
    NW = NC * NS
    CHUNK = B // NW
    NG = CHUNK // 16
    NB = CHUNK // 128

    mesh = plsc.VectorSubcoreMesh(core_axis_name="c", subcore_axis_name="s")

    @functools.partial(
        pl.kernel,
        mesh=mesh,
        out_type=jax.ShapeDtypeStruct((B,), jnp.float32),
        scratch_types=[
            pltpu.VMEM((CHUNK,), jnp.int32),
            pltpu.VMEM((CHUNK,), jnp.int32),
            pltpu.VMEM((CHUNK,), jnp.int32),
            pltpu.VMEM((CHUNK,), jnp.float32),
            pltpu.VMEM((CHUNK,), jnp.float32),
            pltpu.VMEM((16,), jnp.float32),
            pltpu.VMEM((CHUNK,), jnp.float32),
            pltpu.VMEM((CHUNK,), jnp.float32),
            pltpu.VMEM((CHUNK,), jnp.float32),
            pltpu.VMEM((IN_DIM + 8,), jnp.float32),
            pltpu.VMEM((CHUNK,), jnp.float32),
            pltpu.SemaphoreType.DMA,
        ],
        compiler_params=pltpu.CompilerParams(
            needs_layout_passes=False, skip_device_barrier=True),
    )
    def sc_combine(uid, iid, sid, du, di, ctxd, sd, visits, buys, wb,
                   out, uid_v, iid_v, sid_v, dug_v, dig_v, sd_v,
                   ctx_v, vis_v, buy_v, w_v, out_v, sem):
        wid = lax.axis_index("s") * NC + lax.axis_index("c")
        base = wid * CHUNK

        pltpu.sync_copy(uid.at[pl.ds(base, CHUNK)], uid_v)
        pltpu.sync_copy(iid.at[pl.ds(base, CHUNK)], iid_v)

        copies = []
        for k in range(NB):
            sl = pl.ds(k * 128, 128)
            copies.append(pltpu.async_copy(
                du.at[uid_v.at[sl]], dug_v.at[sl], sem))
            copies.append(pltpu.async_copy(
                di.at[iid_v.at[sl]], dig_v.at[sl], sem))

        pltpu.sync_copy(sid.at[pl.ds(base, CHUNK)], sid_v)
        pltpu.sync_copy(sd.at[pl.ds(0, 16)], sd_v)
        pltpu.sync_copy(wb, w_v)
        pltpu.sync_copy(ctxd.at[pl.ds(base, CHUNK)], ctx_v)
        pltpu.sync_copy(visits.at[pl.ds(base, CHUNK)], vis_v)
        pltpu.sync_copy(buys.at[pl.ds(base, CHUNK)], buy_v)

        wvecs = [w_v[pl.ds(c * 16, 16)] for c in range((IN_DIM + 8) // 16)]

        def wscal(j):
            return wvecs[j // 16][j % 16]

        bias = wscal(IN_DIM)

        for c in copies:
            c.wait()

        @pl.loop(0, NG)
        def _group(g):
            off = pl.multiple_of(g * 16, 16)
            sval = plsc.load_gather(sd_v, [sid_v[pl.ds(off, 16)]])
            acc = (bias + dug_v[pl.ds(off, 16)] + dig_v[pl.ds(off, 16)]
                   + sval + ctx_v[pl.ds(off, 16)]
                   + vis_v[pl.ds(off, 16)] * wscal(22)
                   + buy_v[pl.ds(off, 16)] * wscal(23))
            out_v[pl.ds(off, 16)] = 1.0 / (1.0 + jnp.exp(-acc))

        pltpu.sync_copy(out_v, out.at[pl.ds(base, CHUNK)])

    return sc_combine


def kernel(user_ids, shift_ids, item_ids, category, info, visits, buys,
           user_table, item_table, shift_table, W, b):
    del category
    B = user_ids.shape[0]
    uid = user_ids.astype(jnp.int32)
    iid = item_ids.astype(jnp.int32)
    sid = shift_ids.astype(jnp.int32)
    n_info = info.shape[1]
    wb = jnp.concatenate([
        W.reshape(-1).astype(jnp.float32),
        b.reshape(-1).astype(jnp.float32),
        jnp.zeros((7,), jnp.float32),
    ])
    du = _build_rowdot(F, user_table.shape[0], 32768, CTX + F)(
        W, user_table.T)
    di = _build_rowdot(F, item_table.shape[0], 32768, CTX + 2 * F)(
        W, item_table.T)
    ctxd, sd = _build_ctx_shift(n_info, B, shift_table.shape[0])(
        W, info.T, shift_table.T)
    fwd = _build_sc_combine(B)
    out = fwd(uid, iid, sid, du, di, ctxd, sd, visits, buys, wb)
    return out.reshape(B, 1)


# speedup vs baseline: 1.0167x; 1.0167x over previous
import functools

import jax
import jax.numpy as jnp
from jax import lax
from jax.experimental import pallas as pl
from jax.experimental.pallas import tpu as pltpu
from jax.experimental.pallas import tpu_sc as plsc

F = 64
CTX = 24
IN_DIM = 3 * F + CTX


@functools.lru_cache(maxsize=None)
def _build_rowdot(K, N, blk, w_lo):
    nblk = -(-N // blk)

    def body(w_ref, m_ref, o_ref):
        prod = jax.lax.dot_general(
            w_ref[:, w_lo:w_lo + K], m_ref[...], (((1,), (0,)), ((), ())),
            preferred_element_type=jnp.float32,
            precision=jax.lax.Precision.DEFAULT)
        o_ref[...] = prod.reshape(o_ref.shape)

    return pl.pallas_call(
        body,
        grid=(nblk,),
        in_specs=[
            pl.BlockSpec((1, IN_DIM), lambda j: (0, 0)),
            pl.BlockSpec((K, blk), lambda j: (0, j)),
        ],
        out_specs=pl.BlockSpec((blk,), lambda j: (j,)),
        out_shape=jax.ShapeDtypeStruct((nblk * blk,), jnp.float32),
    )


@functools.lru_cache(maxsize=None)
def _build_ctx_shift(n_info, B, n_shift):
    def body(w_ref, m_ref, st_ref, o_ref, sd_ref):
        prod = jax.lax.dot_general(
            w_ref[:, :n_info], m_ref[...], (((1,), (0,)), ((), ())),
            preferred_element_type=jnp.float32,
            precision=jax.lax.Precision.DEFAULT)
        o_ref[...] = prod.reshape(o_ref.shape)
        sdot = jax.lax.dot_general(
            w_ref[:, CTX:CTX + F], st_ref[...], (((1,), (0,)), ((), ())),
            preferred_element_type=jnp.float32,
            precision=jax.lax.Precision.DEFAULT)
        sd_ref[...] = jnp.pad(sdot.reshape(n_shift), (0, 128 - n_shift))

    return pl.pallas_call(
        body,
        in_specs=[
            pl.BlockSpec((1, IN_DIM), lambda: (0, 0)),
            pl.BlockSpec((n_info, B), lambda: (0, 0)),
            pl.BlockSpec((F, n_shift), lambda: (0, 0)),
        ],
        out_specs=[
            pl.BlockSpec((B,), lambda: (0,)),
            pl.BlockSpec((128,), lambda: (0,)),
        ],
        out_shape=[
            jax.ShapeDtypeStruct((B,), jnp.float32),
            jax.ShapeDtypeStruct((128,), jnp.float32),
        ],
    )


@functools.lru_cache(maxsize=None)
def _build_fused_dots(NU, NI, B, n_info, n_shift, blk):
    nu = -(-NU // blk)
    ni = -(-NI // blk)
    last = nu + ni

    def body(w_ref, mu_ref, mi_ref, info_ref, st_ref,
             du_ref, di_ref, ctx_ref, sd_ref):
        j = pl.program_id(0)

        def dot(wlo, wk, m):
            return jax.lax.dot_general(
                w_ref[:, wlo:wlo + wk], m, (((1,), (0,)), ((), ())),
                preferred_element_type=jnp.float32,
                precision=jax.lax.Precision.DEFAULT)

        @pl.when(j < nu)
        def _user():
            du_ref[...] = dot(CTX + F, F, mu_ref[...]).reshape(du_ref.shape)

        @pl.when((j >= nu) & (j < last))
        def _item():
            di_ref[...] = dot(CTX + 2 * F, F, mi_ref[...]).reshape(di_ref.shape)

        @pl.when(j == last)
        def _ctx():
            ctx_ref[...] = dot(0, n_info, info_ref[...]).reshape(ctx_ref.shape)
            sd_ref[...] = jnp.pad(
                dot(CTX, F, st_ref[...]).reshape(n_shift), (0, 128 - n_shift))

    return pl.pallas_call(
        body,
        grid=(last + 1,),
        in_specs=[
            pl.BlockSpec((1, IN_DIM), lambda j: (0, 0)),
            pl.BlockSpec((F, blk), lambda j: (0, jnp.minimum(j, nu - 1))),
            pl.BlockSpec((F, blk),
                         lambda j: (0, jnp.clip(j - nu, 0, ni - 1))),
            pl.BlockSpec((n_info, B), lambda j: (0, 0)),
            pl.BlockSpec((F, n_shift), lambda j: (0, 0)),
        ],
        out_specs=[
            pl.BlockSpec((blk,), lambda j: (jnp.minimum(j, nu - 1),)),
            pl.BlockSpec((blk,), lambda j: (jnp.clip(j - nu, 0, ni - 1),)),
            pl.BlockSpec((B,), lambda j: (0,)),
            pl.BlockSpec((128,), lambda j: (0,)),
        ],
        out_shape=[
            jax.ShapeDtypeStruct((nu * blk,), jnp.float32),
            jax.ShapeDtypeStruct((ni * blk,), jnp.float32),
            jax.ShapeDtypeStruct((B,), jnp.float32),
            jax.ShapeDtypeStruct((128,), jnp.float32),
        ],
    )


@functools.lru_cache(maxsize=None)
def _build_sc_combine(B):
    NC, NS = 2, 16
    NW = NC * NS
    CHUNK = B // NW
    NG = CHUNK // 16
    NB = CHUNK // 128

    mesh = plsc.VectorSubcoreMesh(core_axis_name="c", subcore_axis_name="s")

    @functools.partial(
        pl.kernel,
        mesh=mesh,
        out_type=jax.ShapeDtypeStruct((B,), jnp.float32),
        scratch_types=[
            pltpu.VMEM((CHUNK,), jnp.int32),
            pltpu.VMEM((CHUNK,), jnp.int32),
            pltpu.VMEM((CHUNK,), jnp.int32),
            pltpu.VMEM((CHUNK,), jnp.float32),
            pltpu.VMEM((CHUNK,), jnp.float32),
            pltpu.VMEM((16,), jnp.float32),
            pltpu.VMEM((CHUNK,), jnp.float32),
            pltpu.VMEM((CHUNK,), jnp.float32),
            pltpu.VMEM((CHUNK,), jnp.float32),
            pltpu.VMEM((IN_DIM + 8,), jnp.float32),
            pltpu.VMEM((CHUNK,), jnp.float32),
            pltpu.SemaphoreType.DMA,
        ],
        compiler_params=pltpu.CompilerParams(needs_layout_passes=False),
    )
    def sc_combine(uid, iid, sid, du, di, ctxd, sd, visits, buys, wb,
                   out, uid_v, iid_v, sid_v, dug_v, dig_v, sd_v,
                   ctx_v, vis_v, buy_v, w_v, out_v, sem):
        wid = lax.axis_index("s") * NC + lax.axis_index("c")
        base = wid * CHUNK

        pltpu.sync_copy(uid.at[pl.ds(base, CHUNK)], uid_v)
        pltpu.sync_copy(iid.at[pl.ds(base, CHUNK)], iid_v)

        copies = []
        for k in range(NB):
            sl = pl.ds(k * 128, 128)
            copies.append(pltpu.async_copy(
                du.at[uid_v.at[sl]], dug_v.at[sl], sem))
            copies.append(pltpu.async_copy(
                di.at[iid_v.at[sl]], dig_v.at[sl], sem))

        pltpu.sync_copy(sid.at[pl.ds(base, CHUNK)], sid_v)
        pltpu.sync_copy(sd.at[pl.ds(0, 16)], sd_v)
        pltpu.sync_copy(wb, w_v)
        pltpu.sync_copy(ctxd.at[pl.ds(base, CHUNK)], ctx_v)
        pltpu.sync_copy(visits.at[pl.ds(base, CHUNK)], vis_v)
        pltpu.sync_copy(buys.at[pl.ds(base, CHUNK)], buy_v)

        wvecs = [w_v[pl.ds(c * 16, 16)] for c in range((IN_DIM + 8) // 16)]

        def wscal(j):
            return wvecs[j // 16][j % 16]

        bias = wscal(IN_DIM)

        for c in copies:
            c.wait()

        @pl.loop(0, NG)
        def _group(g):
            off = pl.multiple_of(g * 16, 16)
            sval = plsc.load_gather(sd_v, [sid_v[pl.ds(off, 16)]])
            acc = (bias + dug_v[pl.ds(off, 16)] + dig_v[pl.ds(off, 16)]
                   + sval + ctx_v[pl.ds(off, 16)]
                   + vis_v[pl.ds(off, 16)] * wscal(22)
                   + buy_v[pl.ds(off, 16)] * wscal(23))
            out_v[pl.ds(off, 16)] = 1.0 / (1.0 + jnp.exp(-acc))

        pltpu.sync_copy(out_v, out.at[pl.ds(base, CHUNK)])

    return sc_combine


def kernel(user_ids, shift_ids, item_ids, category, info, visits, buys,
           user_table, item_table, shift_table, W, b):
    del category
    B = user_ids.shape[0]
    uid = user_ids.astype(jnp.int32)
    iid = item_ids.astype(jnp.int32)
    sid = shift_ids.astype(jnp.int32)
    n_info = info.shape[1]
    wb = jnp.concatenate([
        W.reshape(-1).astype(jnp.float32),
        b.reshape(-1).astype(jnp.float32),
        jnp.zeros((7,), jnp.float32),
    ])
    du, di, ctxd, sd = _build_fused_dots(
        user_table.shape[0], item_table.shape[0], B, n_info,
        shift_table.shape[0], 32768)(
        W, user_table.T, item_table.T, info.T, shift_table.T)
    fwd = _build_sc_combine(B)
    out = fwd(uid, iid, sid, du, di, ctxd, sd, visits, buys, wb)
    return out.reshape(B, 1)
